# baseline (speedup 1.0000x reference)
import functools

import jax
import jax.numpy as jnp
from jax import lax
from jax.experimental import pallas as pl
from jax.experimental.pallas import tpu as pltpu

_ROT1 = (13, 15, 26, 6)
_ROT2 = (17, 29, 16, 24)
_KS0 = 0
_KS1 = 42
_KS2 = 0x1BD11BDA ^ _KS0 ^ _KS1


def _rotl(x, r):
    return lax.shift_left(x, jnp.uint32(r)) | lax.shift_right_logical(
        x, jnp.uint32(32 - r))


def _rounds(x0, x1, rots):
    for r in rots:
        x0 = x0 + x1
        x1 = _rotl(x1, r) ^ x0
    return x0, x1


def _threefry_bits(idx):
    x0 = jnp.zeros_like(idx) + jnp.uint32(_KS0)
    x1 = idx + jnp.uint32(_KS1)
    x0, x1 = _rounds(x0, x1, _ROT1)
    x0 = x0 + jnp.uint32(_KS1)
    x1 = x1 + jnp.uint32(_KS2 + 1)
    x0, x1 = _rounds(x0, x1, _ROT2)
    x0 = x0 + jnp.uint32(_KS2)
    x1 = x1 + jnp.uint32(_KS0 + 2)
    x0, x1 = _rounds(x0, x1, _ROT1)
    x0 = x0 + jnp.uint32(_KS0)
    x1 = x1 + jnp.uint32(_KS1 + 3)
    x0, x1 = _rounds(x0, x1, _ROT2)
    x0 = x0 + jnp.uint32(_KS1)
    x1 = x1 + jnp.uint32(_KS2 + 4)
    x0, x1 = _rounds(x0, x1, _ROT1)
    x0 = x0 + jnp.uint32(_KS2)
    x1 = x1 + jnp.uint32(_KS0 + 5)
    return x0 ^ x1


def _gumbel_argmax_kernel(logits_ref, out_ref, best_val, best_idx,
                          *, ncol, block_cols, nblocks):
    j = pl.program_id(0)
    nrow = logits_ref.shape[0]

    cols = lax.broadcasted_iota(jnp.int32, (nrow, block_cols), 1) + j * block_cols
    rows = lax.broadcasted_iota(jnp.uint32, (nrow, block_cols), 0)
    flat = rows * jnp.uint32(ncol) + cols.astype(jnp.uint32)

    bits = _threefry_bits(flat)
    fbits = lax.shift_right_logical(bits, jnp.uint32(9)) | jnp.uint32(0x3F800000)
    u = lax.bitcast_convert_type(fbits, jnp.float32) - jnp.float32(1.0)
    u = jnp.maximum(u + jnp.float32(1e-20), jnp.float32(1e-20))
    g = -jnp.log(-jnp.log(u))

    v = logits_ref[...] + g
    v = jnp.where(cols < ncol, v, -jnp.inf)

    bmax = jnp.max(v, axis=1, keepdims=True)
    bidx = jnp.min(jnp.where(v == bmax, cols, jnp.int32(0x7FFFFFFF)),
                   axis=1, keepdims=True)

    @pl.when(j == 0)
    def _():
        best_val[...] = bmax
        best_idx[...] = bidx

    @pl.when(j > 0)
    def _():
        prev_v = best_val[...]
        prev_i = best_idx[...]
        upd = bmax > prev_v
        best_val[...] = jnp.where(upd, bmax, prev_v)
        best_idx[...] = jnp.where(upd, bidx, prev_i)

    @pl.when(j == nblocks - 1)
    def _():
        out_ref[...] = best_idx[...]


def kernel(logits):
    nrow, ncol = logits.shape
    block_cols = 8192
    nblocks = pl.cdiv(ncol, block_cols)

    body = functools.partial(_gumbel_argmax_kernel, ncol=ncol,
                             block_cols=block_cols, nblocks=nblocks)
    out = pl.pallas_call(
        body,
        grid=(nblocks,),
        in_specs=[pl.BlockSpec((nrow, block_cols), lambda j: (0, j))],
        out_specs=pl.BlockSpec((nrow, 1), lambda j: (0, 0)),
        out_shape=jax.ShapeDtypeStruct((nrow, 1), jnp.int32),
        scratch_shapes=[
            pltpu.VMEM((nrow, 1), jnp.float32),
            pltpu.VMEM((nrow, 1), jnp.int32),
        ],
    )(logits)
    return out.reshape(nrow).astype(jnp.int64)


# speedup vs baseline: 1.4628x; 1.4628x over previous
import functools

import jax
import jax.numpy as jnp
from jax import lax
from jax.experimental import pallas as pl
from jax.experimental.pallas import tpu as pltpu

_ROT1 = (13, 15, 26, 6)
_ROT2 = (17, 29, 16, 24)
_KS1 = 42
_KS2 = 0x1BD11BDA ^ _KS1


def _rotl(x, r):
    return lax.shift_left(x, jnp.uint32(r)) | lax.shift_right_logical(
        x, jnp.uint32(32 - r))


def _round(x0, x1, r):
    x0 = x0 + x1
    x1 = _rotl(x1, r) ^ x0
    return x0, x1


def _threefry_bits(idx):
    x1 = idx + jnp.uint32(_KS1)
    x0 = x1
    x1 = _rotl(x1, _ROT1[0]) ^ x0
    for r in _ROT1[1:]:
        x0, x1 = _round(x0, x1, r)
    x0 = x0 + jnp.uint32(_KS1)
    x1 = x1 + jnp.uint32(_KS2 + 1)
    for r in _ROT2:
        x0, x1 = _round(x0, x1, r)
    x0 = x0 + jnp.uint32(_KS2)
    x1 = x1 + jnp.uint32(2)
    for r in _ROT1:
        x0, x1 = _round(x0, x1, r)
    x1 = x1 + jnp.uint32(_KS1 + 3)
    for r in _ROT2:
        x0, x1 = _round(x0, x1, r)
    x0 = x0 + jnp.uint32(_KS1)
    x1 = x1 + jnp.uint32(_KS2 + 4)
    for r in _ROT1:
        x0, x1 = _round(x0, x1, r)
    x0 = x0 + jnp.uint32(_KS2)
    x1 = x1 + jnp.uint32(5)
    return x0 ^ x1


def _gumbel_argmax_kernel(logits_ref, out_ref, best_val, best_idx,
                          *, ncol, block_cols, slice_cols, nblocks):
    j = pl.program_id(0)
    nrow = logits_ref.shape[0]
    nslices = block_cols // slice_cols

    row_term = lax.broadcasted_iota(jnp.uint32, (nrow, slice_cols), 0) * \
        jnp.uint32(ncol)
    col_iota = lax.broadcasted_iota(jnp.int32, (nrow, slice_cols), 1)
    col0 = j * block_cols

    bv = None
    bi = None
    for s in range(nslices):
        cols = col_iota + (col0 + s * slice_cols)
        flat = row_term + cols.astype(jnp.uint32)
        bits = _threefry_bits(flat)
        fbits = lax.shift_right_logical(bits, jnp.uint32(9)) | \
            jnp.uint32(0x3F800000)
        u = lax.bitcast_convert_type(fbits, jnp.float32) - jnp.float32(1.0)
        u = u + jnp.float32(1e-20)
        nlu = -jnp.log(u)
        v = logits_ref[:, s * slice_cols:(s + 1) * slice_cols] - jnp.log(nlu)
        v = jnp.where(cols < ncol, v, -jnp.inf)
        m = jnp.max(v, axis=1, keepdims=True)
        i = jnp.min(jnp.where(v == m, cols, jnp.int32(0x7FFFFFFF)),
                    axis=1, keepdims=True)
        if s == 0:
            bv, bi = m, i
        else:
            upd = m > bv
            bv = jnp.where(upd, m, bv)
            bi = jnp.where(upd, i, bi)

    @pl.when(j == 0)
    def _():
        best_val[...] = bv
        best_idx[...] = bi

    @pl.when(j > 0)
    def _():
        prev_v = best_val[...]
        prev_i = best_idx[...]
        upd = bv > prev_v
        best_val[...] = jnp.where(upd, bv, prev_v)
        best_idx[...] = jnp.where(upd, bi, prev_i)

    @pl.when(j == nblocks - 1)
    def _():
        out_ref[...] = best_idx[...]


def kernel(logits):
    nrow, ncol = logits.shape
    block_cols = 8192
    slice_cols = 512
    nblocks = pl.cdiv(ncol, block_cols)

    body = functools.partial(_gumbel_argmax_kernel, ncol=ncol,
                             block_cols=block_cols, slice_cols=slice_cols,
                             nblocks=nblocks)
    out = pl.pallas_call(
        body,
        grid=(nblocks,),
        in_specs=[pl.BlockSpec((nrow, block_cols), lambda j: (0, j))],
        out_specs=pl.BlockSpec((nrow, 1), lambda j: (0, 0)),
        out_shape=jax.ShapeDtypeStruct((nrow, 1), jnp.int32),
        scratch_shapes=[
            pltpu.VMEM((nrow, 1), jnp.float32),
            pltpu.VMEM((nrow, 1), jnp.int32),
        ],
    )(logits)
    return out.reshape(nrow).astype(jnp.int64)
